# baseline (speedup 1.0000x reference)
import functools

import jax
import jax.numpy as jnp
from jax import lax
from jax.experimental import pallas as pl
from jax.experimental.pallas import tpu as pltpu
from jax.experimental.pallas import tpu_sc as plsc

N_INDICATORS = 8
HIDDEN = 1024
ROWS = 4096 * 20
NUM_CORES = 2
NUM_SUBCORES = 16
NW = NUM_CORES * NUM_SUBCORES
B_PER_W = ROWS // NW
CROWS = 40
N_CHUNKS = B_PER_W // CROWS


def _sc_lookup(table, ids3):
    mesh = plsc.VectorSubcoreMesh(core_axis_name="c", subcore_axis_name="s")

    @functools.partial(
        pl.kernel,
        mesh=mesh,
        out_type=jax.ShapeDtypeStruct((NW, B_PER_W, HIDDEN), jnp.float32),
        scratch_types=[
            pltpu.VMEM((N_CHUNKS, CROWS), jnp.int32),
            pltpu.VMEM((CROWS, HIDDEN), jnp.float32),
            pltpu.VMEM((CROWS, HIDDEN), jnp.float32),
            pltpu.SemaphoreType.DMA,
            pltpu.SemaphoreType.DMA,
            pltpu.SemaphoreType.DMA,
        ],
    )
    def k(table_hbm, ids_hbm, out_hbm, idx_v, buf0, buf1, gsem, sem0, sem1):
        wid = lax.axis_index("s") * NUM_CORES + lax.axis_index("c")
        out_w = out_hbm.at[wid]
        pltpu.sync_copy(ids_hbm.at[wid], idx_v)

        def step(t, carry):
            for b, buf, sem in ((0, buf0, sem0), (1, buf1, sem1)):
                j = 2 * t + b

                pltpu.async_copy(table_hbm.at[idx_v.at[j]], buf, gsem).wait()
            return carry

        lax.fori_loop(0, N_CHUNKS // 2, step, 0)

    return k(table, ids3)


def kernel(ids, indicator_embs):
    ids3 = ids.reshape(NW, N_CHUNKS, CROWS).astype(jnp.int32)
    out = _sc_lookup(indicator_embs, ids3)
    return out.reshape(4096, 20, HIDDEN)


# speedup vs baseline: 1.3891x; 1.3891x over previous
import functools

import jax
import jax.numpy as jnp
from jax import lax
from jax.experimental import pallas as pl
from jax.experimental.pallas import tpu as pltpu
from jax.experimental.pallas import tpu_sc as plsc

N_INDICATORS = 8
HIDDEN = 1024
ROWS = 4096 * 20
NUM_CORES = 2
NUM_SUBCORES = 16
NW = NUM_CORES * NUM_SUBCORES
PAIRS_PER_W = ROWS // 2 // NW
CP = 16
N_CHUNKS = PAIRS_PER_W // CP


def _sc_lookup(ptable, ev3, od3):
    mesh = plsc.VectorSubcoreMesh(core_axis_name="c", subcore_axis_name="s")

    @functools.partial(
        pl.kernel,
        mesh=mesh,
        compiler_params=pltpu.CompilerParams(needs_layout_passes=False),
        out_type=jax.ShapeDtypeStruct((NW, PAIRS_PER_W, 2 * HIDDEN), jnp.float32),
        scratch_types=[
            pltpu.VMEM((N_CHUNKS, CP), jnp.int32),
            pltpu.VMEM((N_CHUNKS, CP), jnp.int32),
            pltpu.VMEM((CP, 2 * HIDDEN), jnp.float32),
            pltpu.VMEM((CP, 2 * HIDDEN), jnp.float32),
            pltpu.SemaphoreType.DMA,
            pltpu.SemaphoreType.DMA,
            pltpu.SemaphoreType.DMA,
        ],
    )
    def k(pt_hbm, ev_hbm, od_hbm, out_hbm, ev_v, od_v, buf0, buf1,
          gsem, sem0, sem1):
        wid = lax.axis_index("s") * NUM_CORES + lax.axis_index("c")
        out_w = out_hbm.at[wid]
        pltpu.sync_copy(ev_hbm.at[wid], ev_v)
        pltpu.sync_copy(od_hbm.at[wid], od_v)

        def step(t, carry):
            for b, buf, sem in ((0, buf0, sem0), (1, buf1, sem1)):
                j = 2 * t + b

                @pl.when(t >= 1)
                def _wait(buf=buf, sem=sem):
                    pltpu.make_async_copy(
                        buf, out_w.at[pl.ds(0, CP)], sem
                    ).wait()

                pid = ev_v[j] * N_INDICATORS + od_v[j]
                pltpu.async_copy(pt_hbm.at[pid], buf, gsem).wait()
                pltpu.async_copy(buf, out_w.at[pl.ds(j * CP, CP)], sem)
            return carry

        lax.fori_loop(0, N_CHUNKS // 2, step, 0)
        pltpu.make_async_copy(buf0, out_w.at[pl.ds(0, CP)], sem0).wait()
        pltpu.make_async_copy(buf1, out_w.at[pl.ds(0, CP)], sem1).wait()

    return k(ptable, ev3, od3)


def kernel(ids, indicator_embs):
    ids_flat = ids.reshape(-1).astype(jnp.int32)
    ev3 = ids_flat[0::2].reshape(NW, N_CHUNKS, CP)
    od3 = ids_flat[1::2].reshape(NW, N_CHUNKS, CP)
    ptable = jnp.concatenate(
        [
            jnp.repeat(indicator_embs, N_INDICATORS, axis=0),
            jnp.tile(indicator_embs, (N_INDICATORS, 1)),
        ],
        axis=1,
    )
    out = _sc_lookup(ptable, ev3, od3)
    return out.reshape(4096, 20, HIDDEN)
